# baseline (speedup 1.0000x reference)
import functools

import jax
import jax.numpy as jnp
from jax import lax
from jax.experimental import pallas as pl
from jax.experimental.pallas import tpu as pltpu
from jax.experimental.pallas import tpu_sc as plsc

_NC = 2
_NS = 16
_NW = _NC * _NS
_ICHUNK = 128


def _fused_body(x_ref, w_ref, res_ref, idx_ref):
    w = w_ref[...]
    d, k = w.shape
    esq = jnp.sum(w * w, axis=0)[:, None]
    w_hi = w.astype(jnp.bfloat16)
    r1 = w - w_hi.astype(jnp.float32)
    w_mid = r1.astype(jnp.bfloat16)
    w_lo = (r1 - w_mid.astype(jnp.float32)).astype(jnp.bfloat16)
    w3 = jnp.concatenate([w_hi, w_mid, w_lo], axis=0)
    ss = [
        lax.dot_general(
            w, x_ref[j], (((0,), (0,)), ((), ())),
            preferred_element_type=jnp.float32,
        )
        for j in range(x_ref.shape[0])
    ]
    for j in range(x_ref.shape[0]):
        x = x_ref[j]
        xsq = jnp.sum(x * x, axis=0)[None, :]
        s = ss[j]
        dist = (xsq - 2.0 * s) + esq
        idx = jnp.argmin(dist, axis=0).astype(jnp.int32)
        idx_ref[j, 0, :] = idx
        onehot = (
            lax.broadcasted_iota(jnp.int32, (k, 1), 0) == idx[None, :]
        ).astype(jnp.bfloat16)
        res3 = lax.dot_general(
            w3, onehot, (((1,), (0,)), ((), ())),
            preferred_element_type=jnp.float32,
        )
        res_ref[j] = (res3[:d] + res3[d:2 * d]) + res3[2 * d:]


def _fused_nearest(x3, weight):
    b, d, m = x3.shape
    k = weight.shape[1]
    bpb = b
    res, idx3 = pl.pallas_call(
        _fused_body,
        grid=(b // bpb,),
        in_specs=[
            pl.BlockSpec((bpb, d, m), lambda i: (i, 0, 0)),
            pl.BlockSpec((d, k), lambda i: (0, 0)),
        ],
        out_specs=[
            pl.BlockSpec((bpb, d, m), lambda i: (i, 0, 0)),
            pl.BlockSpec((bpb, 1, m), lambda i: (i, 0, 0)),
        ],
        out_shape=[
            jax.ShapeDtypeStruct((b, d, m), jnp.float32),
            jax.ShapeDtypeStruct((b, 1, m), jnp.int32),
        ],
    )(x3, weight)
    return res, idx3


def _sc_gather(table, idx, n, d):
    bpw = n // _NW
    nchunk = bpw // _ICHUNK
    idx3 = idx.reshape(_NW, nchunk, _ICHUNK)
    mesh = plsc.VectorSubcoreMesh(core_axis_name="c", subcore_axis_name="s")

    @functools.partial(
        pl.kernel,
        mesh=mesh,
        out_type=jax.ShapeDtypeStruct((_NW, nchunk, _ICHUNK, d), jnp.float32),
        scratch_types=[
            pltpu.VMEM((nchunk, _ICHUNK), jnp.int32),
            pltpu.VMEM((nchunk, _ICHUNK, d), jnp.float32),
            pltpu.SemaphoreType.DMA,
        ],
        compiler_params=pltpu.CompilerParams(use_tc_tiling_on_sc=False),
    )
    def gather_kernel(table_hbm, idx_hbm, out_hbm, idx_v, rows_v, sem):
        wid = lax.axis_index("s") * _NC + lax.axis_index("c")
        pltpu.sync_copy(idx_hbm.at[wid], idx_v)
        copies = [
            pltpu.async_copy(table_hbm.at[idx_v.at[j]], rows_v.at[j], sem)
            for j in range(nchunk)
        ]
        for c in copies:
            c.wait()
        pltpu.sync_copy(rows_v, out_hbm.at[wid])

    return gather_kernel(table, idx3).reshape(n, d)


def kernel(x, weight):
    b, d, h, w = x.shape
    res, idx3 = _fused_nearest(x.reshape(b, d, h * w), weight)
    return res.reshape(b, d, h, w), idx3.reshape(b, h, w)


# speedup vs baseline: 1.0638x; 1.0638x over previous
import functools

import jax
import jax.numpy as jnp
from jax import lax
from jax.experimental import pallas as pl
from jax.experimental.pallas import tpu as pltpu
from jax.experimental.pallas import tpu_sc as plsc

_NC = 2
_NS = 16
_NW = _NC * _NS
_ICHUNK = 128


def _fused_body(x_ref, w_ref, res_ref, idx_ref):
    w = w_ref[...]
    d, k = w.shape
    esq = jnp.sum(w * w, axis=0)[:, None]
    w_hi = w.astype(jnp.bfloat16)
    r1 = w - w_hi.astype(jnp.float32)
    w_mid = r1.astype(jnp.bfloat16)
    w_lo = (r1 - w_mid.astype(jnp.float32)).astype(jnp.bfloat16)
    w3 = jnp.concatenate([w_hi, w_mid, w_lo], axis=0)
    w2 = w * -2.0
    ss = [
        lax.dot_general(
            w2, x_ref[j], (((0,), (0,)), ((), ())),
            preferred_element_type=jnp.float32,
        )
        for j in range(x_ref.shape[0])
    ]
    for j in range(x_ref.shape[0]):
        x = x_ref[j]
        xsq = jnp.sum(x * x, axis=0)[None, :]
        s = ss[j]
        dist = (xsq + s) + esq
        idx = jnp.argmin(dist, axis=0).astype(jnp.int32)
        idx_ref[j, 0, :] = idx
        onehot = (
            lax.broadcasted_iota(jnp.int32, (k, 1), 0) == idx[None, :]
        ).astype(jnp.bfloat16)
        res3 = lax.dot_general(
            w3, onehot, (((1,), (0,)), ((), ())),
            preferred_element_type=jnp.float32,
        )
        res_ref[j] = (res3[:d] + res3[d:2 * d]) + res3[2 * d:]


def _fused_nearest(x3, weight):
    b, d, m = x3.shape
    k = weight.shape[1]
    bpb = 4 if b % 4 == 0 else 1
    res, idx3 = pl.pallas_call(
        _fused_body,
        grid=(b // bpb,),
        in_specs=[
            pl.BlockSpec((bpb, d, m), lambda i: (i, 0, 0)),
            pl.BlockSpec((d, k), lambda i: (0, 0)),
        ],
        out_specs=[
            pl.BlockSpec((bpb, d, m), lambda i: (i, 0, 0)),
            pl.BlockSpec((bpb, 1, m), lambda i: (i, 0, 0)),
        ],
        out_shape=[
            jax.ShapeDtypeStruct((b, d, m), jnp.float32),
            jax.ShapeDtypeStruct((b, 1, m), jnp.int32),
        ],
    )(x3, weight)
    return res, idx3


def _sc_gather(table, idx, n, d):
    bpw = n // _NW
    nchunk = bpw // _ICHUNK
    idx3 = idx.reshape(_NW, nchunk, _ICHUNK)
    mesh = plsc.VectorSubcoreMesh(core_axis_name="c", subcore_axis_name="s")

    @functools.partial(
        pl.kernel,
        mesh=mesh,
        out_type=jax.ShapeDtypeStruct((_NW, nchunk, _ICHUNK, d), jnp.float32),
        scratch_types=[
            pltpu.VMEM((nchunk, _ICHUNK), jnp.int32),
            pltpu.VMEM((nchunk, _ICHUNK, d), jnp.float32),
            pltpu.SemaphoreType.DMA,
        ],
        compiler_params=pltpu.CompilerParams(use_tc_tiling_on_sc=False),
    )
    def gather_kernel(table_hbm, idx_hbm, out_hbm, idx_v, rows_v, sem):
        wid = lax.axis_index("s") * _NC + lax.axis_index("c")
        pltpu.sync_copy(idx_hbm.at[wid], idx_v)
        copies = [
            pltpu.async_copy(table_hbm.at[idx_v.at[j]], rows_v.at[j], sem)
            for j in range(nchunk)
        ]
        for c in copies:
            c.wait()
        pltpu.sync_copy(rows_v, out_hbm.at[wid])

    return gather_kernel(table, idx3).reshape(n, d)


def kernel(x, weight):
    b, d, h, w = x.shape
    res, idx3 = _fused_nearest(x.reshape(b, d, h * w), weight)
    return res.reshape(b, d, h, w), idx3.reshape(b, h, w)
